# baseline (speedup 1.0000x reference)
import functools

import jax
import jax.numpy as jnp
from jax import lax
from jax.experimental import pallas as pl
from jax.experimental.pallas import tpu as pltpu
from jax.experimental.pallas import tpu_sc as plsc

KNB = 16


def _topk_body(pts_blk, pts_all, idx_ref, colf_ref, *, R, N, B0):
    b = pl.program_id(0) + B0
    p = pts_blk[0]
    q = pts_all[0]
    rp = jnp.sum(p * p, axis=1, keepdims=True)
    rq = jnp.sum(q * q, axis=1)[None, :]
    m = lax.dot_general(p, q, (((1,), (1,)), ((), ())),
                        preferred_element_type=jnp.float32)
    vals = (rp - 2.0 * m) + rq

    boff = b.astype(jnp.int32) * N
    colf_ref[...] = lax.broadcasted_iota(jnp.int32, (R, N), 1).astype(jnp.float32)
    big = jnp.float32(N)
    rb = pl.program_id(1)
    rowf = (lax.broadcasted_iota(jnp.int32, (R, 1), 0)
            + rb.astype(jnp.int32) * R).astype(jnp.float32)
    vals = jnp.where(colf_ref[...] == rowf, jnp.float32(jnp.inf), vals)
    for k in range(KNB):
        mn = jnp.min(vals, axis=1, keepdims=True)
        first = jnp.min(jnp.where(vals <= mn, colf_ref[...], big), axis=1)
        idx_ref[0, k, :] = first.astype(jnp.int32) + boff
        vals = jnp.where(colf_ref[...] == first[:, None], jnp.float32(jnp.inf), vals)


def _topk_indices(points, b0=0):
    B, N, PD = points.shape
    R = 128
    return pl.pallas_call(
        functools.partial(_topk_body, R=R, N=N, B0=b0),
        grid=(B, N // R),
        in_specs=[
            pl.BlockSpec((1, R, PD), lambda b, rb: (b, rb, 0)),
            pl.BlockSpec((1, N, PD), lambda b, rb: (b, 0, 0)),
        ],
        out_specs=pl.BlockSpec((1, KNB, R), lambda b, rb: (b, 0, rb)),
        out_shape=jax.ShapeDtypeStruct((B, KNB, N), jnp.int32),
        scratch_shapes=[pltpu.VMEM((R, N), jnp.float32)],
    )(points, points)


def _sc_gather(table, idxt, b0=0):
    BN, F = table.shape
    B, K, N = idxt.shape
    info = plsc.get_sparse_core_info()
    NC, NS = info.num_cores, info.num_subcores
    NW = NC * NS
    KPW = (B * K) // NW
    P = 128
    NCHUNK = N // P
    WPB = NW // B

    mesh = plsc.VectorSubcoreMesh(core_axis_name="c", subcore_axis_name="s")

    @functools.partial(
        pl.kernel,
        mesh=mesh,
        out_type=jax.ShapeDtypeStruct((B, N, K, 2 * F), jnp.float32),
        scratch_types=[
            pltpu.VMEM((P,), jnp.int32),
            pltpu.VMEM((2, P, F), jnp.float32),
            pltpu.VMEM((P, F), jnp.float32),
            pltpu.VMEM((2, P, F), jnp.float32),
            pltpu.SemaphoreType.DMA,
            pltpu.SemaphoreType.DMA,
            pltpu.SemaphoreType.DMA,
        ],
    )
    def run(table_hbm, idxt_hbm, out_hbm, idx_v, g_v, c_v, d_v,
            sem_g, sem_w0, sem_w1):
        wid = lax.axis_index("s") * NC + lax.axis_index("c")
        b = wid // WPB
        kg = wid % WPB
        sem_w = (sem_w0, sem_w1)

        def drain_writes(p):
            pltpu.make_async_copy(
                g_v.at[p], out_hbm.at[b, pl.ds(0, P), 0, pl.ds(0, F)],
                sem_w[p]).wait()
            pltpu.make_async_copy(
                d_v.at[p], out_hbm.at[b, pl.ds(0, P), 0, pl.ds(F, F)],
                sem_w[p]).wait()

        def chunk(ci, carry):
            i0 = ci * P
            pltpu.sync_copy(table_hbm.at[pl.ds((b0 + b) * N + i0, P)], c_v)
            for dk in range(KPW):
                k = kg * KPW + dk
                p = dk % 2
                gp, dp = g_v.at[p], d_v.at[p]

                @pl.when(ci * KPW + dk >= 2)
                def _():
                    drain_writes(p)

                pltpu.sync_copy(idxt_hbm.at[b, k, pl.ds(i0, P)], idx_v)
                pltpu.async_copy(table_hbm.at[idx_v], gp, sem_g).wait()

                def row(r, c2):
                    for v in range(F // 16):
                        sl = pl.ds(v * 16, 16)
                        dp[r, sl] = gp[r, sl] - c_v[r, sl]
                    return c2

                lax.fori_loop(0, P, row, 0)
                pltpu.async_copy(gp, out_hbm.at[b, pl.ds(i0, P), k, pl.ds(0, F)],
                                 sem_w[p])
                pltpu.async_copy(dp, out_hbm.at[b, pl.ds(i0, P), k, pl.ds(F, F)],
                                 sem_w[p])
            return carry

        lax.fori_loop(0, NCHUNK, chunk, 0)
        for p in range(2):
            drain_writes(p)

    return run(table, idxt)


def kernel(points, features):
    B, N, _ = points.shape
    F = features.shape[-1]
    H = B // 2
    table = features.reshape(B * N, F)
    idx1 = _topk_indices(points[:H], 0)
    out1 = _sc_gather(table, idx1, 0)
    idx2 = _topk_indices(points[H:], H)
    out2 = _sc_gather(table, idx2, H)
    return jnp.concatenate([out1, out2], axis=0)


# speedup vs baseline: 1.1265x; 1.1265x over previous
import functools

import jax
import jax.numpy as jnp
from jax import lax
from jax.experimental import pallas as pl
from jax.experimental.pallas import tpu as pltpu
from jax.experimental.pallas import tpu_sc as plsc

KNB = 16


def _topk_body(pts_blk, pts_all, idx_ref, colf_ref, *, R, N, B0):
    b = pl.program_id(0) + B0
    p = pts_blk[0]
    q = pts_all[0]
    rp = jnp.sum(p * p, axis=1, keepdims=True)
    rq = jnp.sum(q * q, axis=1)[None, :]
    m = lax.dot_general(p, q, (((1,), (1,)), ((), ())),
                        preferred_element_type=jnp.float32)
    vals = (rp - 2.0 * m) + rq

    boff = b.astype(jnp.int32) * N
    colf_ref[...] = lax.broadcasted_iota(jnp.int32, (R, N), 1).astype(jnp.float32)
    big = jnp.float32(N)
    rb = pl.program_id(1)
    rowf = (lax.broadcasted_iota(jnp.int32, (R, 1), 0)
            + rb.astype(jnp.int32) * R).astype(jnp.float32)
    vals = jnp.where(colf_ref[...] == rowf, jnp.float32(jnp.inf), vals)
    for k in range(KNB):
        mn = jnp.min(vals, axis=1, keepdims=True)
        first = jnp.min(jnp.where(vals <= mn, colf_ref[...], big), axis=1)
        idx_ref[0, k, :] = first.astype(jnp.int32) + boff
        vals = jnp.where(colf_ref[...] == first[:, None], jnp.float32(jnp.inf), vals)


def _topk_indices(points, b0=0):
    B, N, PD = points.shape
    R = 128
    return pl.pallas_call(
        functools.partial(_topk_body, R=R, N=N, B0=b0),
        grid=(B, N // R),
        in_specs=[
            pl.BlockSpec((1, R, PD), lambda b, rb: (b, rb, 0)),
            pl.BlockSpec((1, N, PD), lambda b, rb: (b, 0, 0)),
        ],
        out_specs=pl.BlockSpec((1, KNB, R), lambda b, rb: (b, 0, rb)),
        out_shape=jax.ShapeDtypeStruct((B, KNB, N), jnp.int32),
        scratch_shapes=[pltpu.VMEM((R, N), jnp.float32)],
    )(points, points)


def _sc_gather(table, idxt, b0=0):
    BN, F = table.shape
    B, K, N = idxt.shape
    info = plsc.get_sparse_core_info()
    NC, NS = info.num_cores, info.num_subcores
    NW = NC * NS
    KPW = (B * K) // NW
    P = 128
    NCHUNK = N // P
    WPB = NW // B

    mesh = plsc.VectorSubcoreMesh(core_axis_name="c", subcore_axis_name="s")

    @functools.partial(
        pl.kernel,
        mesh=mesh,
        out_type=jax.ShapeDtypeStruct((B, N, K, 2 * F), jnp.float32),
        scratch_types=[
            pltpu.VMEM((KPW, N), jnp.int32),
            pltpu.VMEM((2, P, F), jnp.float32),
            pltpu.VMEM((P, F), jnp.float32),
            pltpu.VMEM((2, P, F), jnp.float32),
            pltpu.SemaphoreType.DMA,
            pltpu.SemaphoreType.DMA,
            pltpu.SemaphoreType.DMA,
        ],
    )
    def run(table_hbm, idxt_hbm, out_hbm, idx_v, g_v, c_v, d_v,
            sem_g, sem_w0, sem_w1):
        wid = lax.axis_index("s") * NC + lax.axis_index("c")
        b = wid // WPB
        kg = wid % WPB
        sem_w = (sem_w0, sem_w1)
        pltpu.sync_copy(idxt_hbm.at[b, pl.ds(kg * KPW, KPW), :], idx_v)

        def drain_writes(p):
            pltpu.make_async_copy(
                g_v.at[p], out_hbm.at[b, pl.ds(0, P), 0, pl.ds(0, F)],
                sem_w[p]).wait()
            pltpu.make_async_copy(
                d_v.at[p], out_hbm.at[b, pl.ds(0, P), 0, pl.ds(F, F)],
                sem_w[p]).wait()

        def chunk(ci, carry):
            i0 = ci * P
            pltpu.sync_copy(table_hbm.at[pl.ds((b0 + b) * N + i0, P)], c_v)
            for dk in range(KPW):
                k = kg * KPW + dk
                p = dk % 2
                gp, dp = g_v.at[p], d_v.at[p]

                @pl.when(ci * KPW + dk >= 2)
                def _():
                    drain_writes(p)

                pltpu.async_copy(
                    table_hbm.at[idx_v.at[dk, pl.ds(i0, P)]], gp, sem_g).wait()

                def row(r, c2):
                    for v in range(F // 16):
                        sl = pl.ds(v * 16, 16)
                        dp[r, sl] = gp[r, sl] - c_v[r, sl]
                    return c2

                lax.fori_loop(0, P, row, 0)
                pltpu.async_copy(gp, out_hbm.at[b, pl.ds(i0, P), k, pl.ds(0, F)],
                                 sem_w[p])
                pltpu.async_copy(dp, out_hbm.at[b, pl.ds(i0, P), k, pl.ds(F, F)],
                                 sem_w[p])
            return carry

        lax.fori_loop(0, NCHUNK, chunk, 0)
        for p in range(2):
            drain_writes(p)

    return run(table, idxt)


def kernel(points, features):
    B, N, _ = points.shape
    F = features.shape[-1]
    table = features.reshape(B * N, F)
    idxt = _topk_indices(points)
    return _sc_gather(table, idxt)
